# baseline (speedup 1.0000x reference)
import functools

import jax
import jax.numpy as jnp
from jax import lax
from jax.experimental import pallas as pl
from jax.experimental.pallas import tpu as pltpu
from jax.experimental.pallas import tpu_sc as plsc

_B, _H, _W = 16, 384, 384
_POS_WEIGHT = 2.0
_ROWS = _B * _H
_NC, _NS, _L = 2, 16, 16
_NW = _NC * _NS

_K = 6
_TC_IMGS = _B - _K
_TC_ROWS = _TC_IMGS * _H
_TROWS = _K * _H // _NW
_CROWS = 24
_NCHUNK = _TROWS // _CROWS
_CVECS = _CROWS * _W // _L

_BR = 2 * _H
_TC_GRID = _TC_ROWS // _BR

_C0 = 0.99988787
_C1 = -0.49636774
_C2 = 0.30467086
_C3 = -0.15602694
_C4 = 0.04106407

_mesh = plsc.VectorSubcoreMesh(
    core_axis_name="c", subcore_axis_name="s", num_cores=_NC)


@functools.partial(
    pl.kernel,
    mesh=_mesh,
    out_type=jax.ShapeDtypeStruct((2, _NW, _L), jnp.float32),
    scratch_types=[
        pltpu.VMEM((2, _CROWS, _W), jnp.float32),
        pltpu.VMEM((2, _CROWS, _W), jnp.float32),
        pltpu.VMEM((2, _CROWS, _W), jnp.float32),
        pltpu.VMEM((_L,), jnp.int32),
        pltpu.VMEM((_L,), jnp.float32),
        pltpu.SemaphoreType.DMA,
        pltpu.SemaphoreType.DMA,
        pltpu.SemaphoreType.DMA,
        pltpu.SemaphoreType.DMA,
        pltpu.SemaphoreType.DMA,
        pltpu.SemaphoreType.DMA,
    ],
)
def _sc_partial(x_hbm, p_hbm, n_hbm, lbl_hbm, out_hbm, xb, pb, nb, lbv, av,
                sx0, sp0, sn0, sx1, sp1, sn1):
    wid = lax.axis_index("c") * _NS + lax.axis_index("s")
    base = _TC_ROWS + wid * _TROWS
    pltpu.sync_copy(lbl_hbm, lbv)
    lbl = lbv[...]
    sems = ((sx0, sp0, sn0), (sx1, sp1, sn1))

    def issue(ci, b):
        sl = pl.ds(base + ci * _CROWS, _CROWS)
        return (
            pltpu.async_copy(x_hbm.at[sl, :], xb.at[b], sems[b][0]),
            pltpu.async_copy(p_hbm.at[sl, :], pb.at[b], sems[b][1]),
            pltpu.async_copy(n_hbm.at[sl, :], nb.at[b], sems[b][2]),
        )

    def compute(ci, b, carry):
        img = (base + ci * _CROWS) // _H
        iv = jnp.full((_L,), img, jnp.int32)
        lv = lbl.at[iv].get(mode="promise_in_bounds")
        is_pos = lv == 1
        sgn = jnp.where(is_pos, -1.0, 1.0)
        wgt = jnp.where(is_pos, _POS_WEIGHT, 1.0)

        def inner(i, c2):
            acc2, cnt2 = c2
            r = i // (_W // _L)
            sl = pl.ds((i % (_W // _L)) * _L, _L)
            x = xb[b, r, sl]
            p = pb[b, r, sl]
            n = nb[b, r, sl]
            t = x * sgn
            a = jnp.abs(t)
            rl = jnp.maximum(t, 0.0)
            u = jnp.exp(-a)
            poly = _C4
            for c in (_C3, _C2, _C1, _C0):
                poly = poly * u + c
            sp = rl + poly * u
            m = jnp.minimum(p, n) > 0.5
            acc2 = acc2 + jnp.where(m, sp, 0.0)
            cnt2 = cnt2 + jnp.where(m, 1.0, 0.0)
            return acc2, cnt2

        zero = jnp.zeros((_L,), jnp.float32)
        ca, cc = lax.fori_loop(0, _CVECS, inner, (zero, zero))
        acc, cnt = carry
        return acc + ca * wgt, cnt + cc

    zero = jnp.zeros((_L,), jnp.float32)
    acc, cnt = zero, zero
    pend = issue(0, 0)
    for ci in range(_NCHUNK):
        b = ci % 2
        nxt = issue(ci + 1, 1 - b) if ci + 1 < _NCHUNK else None
        for h in pend:
            h.wait()
        acc, cnt = compute(ci, b, (acc, cnt))
        pend = nxt
    av[...] = acc
    pltpu.sync_copy(av, out_hbm.at[0, wid])
    av[...] = cnt
    pltpu.sync_copy(av, out_hbm.at[1, wid])


def _tc_body(lbl_ref, x_ref, p_ref, n_ref, os_ref, oc_ref):
    i = pl.program_id(0)
    s_blk = 0.0
    c_blk = 0.0
    for h in range(2):
        y = lbl_ref[i * 2 + h]
        sgn = jnp.where(y == 1, -1.0, 1.0)
        wgt = jnp.where(y == 1, _POS_WEIGHT, 1.0)
        rows = pl.ds(h * _H, _H)
        x = x_ref[rows, :]
        t = x * sgn
        sp = jnp.maximum(t, 0.0) + jnp.log1p(jnp.exp(-jnp.abs(t)))
        m = (p_ref[rows, :] > 0.5) & (n_ref[rows, :] > 0.5)
        s_blk += jnp.sum(jnp.where(m, sp, 0.0)) * wgt
        c_blk += jnp.sum(jnp.where(m, 1.0, 0.0))

    @pl.when(i == 0)
    def _():
        os_ref[0, 0] = 0.0
        oc_ref[0, 0] = 0.0

    os_ref[0, 0] += s_blk
    oc_ref[0, 0] += c_blk


_tc_partial = pl.pallas_call(
    _tc_body,
    grid_spec=pltpu.PrefetchScalarGridSpec(
        num_scalar_prefetch=1,
        grid=(_TC_GRID,),
        in_specs=[
            pl.BlockSpec((_BR, _W), lambda i, *_: (i, 0)),
            pl.BlockSpec((_BR, _W), lambda i, *_: (i, 0)),
            pl.BlockSpec((_BR, _W), lambda i, *_: (i, 0)),
        ],
        out_specs=[
            pl.BlockSpec(memory_space=pltpu.SMEM),
            pl.BlockSpec(memory_space=pltpu.SMEM),
        ],
    ),
    out_shape=[
        jax.ShapeDtypeStruct((1, 1), jnp.float32),
        jax.ShapeDtypeStruct((1, 1), jnp.float32),
    ],
)


def _combine_body(parts_ref, ts_ref, tc_ref, out_ref):
    ps = parts_ref[...]
    num = jnp.sum(ps[0]) + ts_ref[0, 0]
    den = jnp.sum(ps[1]) + tc_ref[0, 0]
    out_ref[0, 0] = num / den


_combine = pl.pallas_call(
    _combine_body,
    in_specs=[
        pl.BlockSpec((2, _NW, _L), lambda: (0, 0, 0)),
        pl.BlockSpec(memory_space=pltpu.SMEM),
        pl.BlockSpec(memory_space=pltpu.SMEM),
    ],
    out_specs=pl.BlockSpec(memory_space=pltpu.SMEM),
    out_shape=jax.ShapeDtypeStruct((1, 1), jnp.float32),
)


def kernel(cancer_logits, prostate_mask, needle_mask, label, involvement):
    del involvement
    x = cancer_logits.reshape(_ROWS, _W)
    p = prostate_mask.reshape(_ROWS, _W)
    n = needle_mask.reshape(_ROWS, _W)
    lbl = label.astype(jnp.int32)
    parts = _sc_partial(x, p, n, lbl)
    tc_s, tc_c = _tc_partial(lbl, x, p, n)
    loss = _combine(parts, tc_s, tc_c)
    return loss[0, 0]


# speedup vs baseline: 1.0808x; 1.0808x over previous
import functools

import jax
import jax.numpy as jnp
from jax import lax
from jax.experimental import pallas as pl
from jax.experimental.pallas import tpu as pltpu
from jax.experimental.pallas import tpu_sc as plsc

_B, _H, _W = 16, 384, 384
_POS_WEIGHT = 2.0
_ROWS = _B * _H
_NC, _NS, _L = 2, 16, 16
_NW = _NC * _NS

_K = 4
_TC_IMGS = _B - _K
_TC_ROWS = _TC_IMGS * _H
_TROWS = _K * _H // _NW
_CROWS = 24
_NCHUNK = _TROWS // _CROWS
_CVECS = _CROWS * _W // _L

_BR = 2 * _H
_TC_GRID = _TC_ROWS // _BR

_C0 = 0.99988787
_C1 = -0.49636774
_C2 = 0.30467086
_C3 = -0.15602694
_C4 = 0.04106407

_mesh = plsc.VectorSubcoreMesh(
    core_axis_name="c", subcore_axis_name="s", num_cores=_NC)


@functools.partial(
    pl.kernel,
    mesh=_mesh,
    out_type=jax.ShapeDtypeStruct((2, _NW, _L), jnp.float32),
    scratch_types=[
        pltpu.VMEM((2, _CROWS, _W), jnp.float32),
        pltpu.VMEM((2, _CROWS, _W), jnp.float32),
        pltpu.VMEM((2, _CROWS, _W), jnp.float32),
        pltpu.VMEM((_L,), jnp.int32),
        pltpu.VMEM((_L,), jnp.float32),
        pltpu.SemaphoreType.DMA,
        pltpu.SemaphoreType.DMA,
        pltpu.SemaphoreType.DMA,
        pltpu.SemaphoreType.DMA,
        pltpu.SemaphoreType.DMA,
        pltpu.SemaphoreType.DMA,
    ],
)
def _sc_partial(x_hbm, p_hbm, n_hbm, lbl_hbm, out_hbm, xb, pb, nb, lbv, av,
                sx0, sp0, sn0, sx1, sp1, sn1):
    wid = lax.axis_index("c") * _NS + lax.axis_index("s")
    base = _TC_ROWS + wid * _TROWS
    pltpu.sync_copy(lbl_hbm, lbv)
    lbl = lbv[...]
    sems = ((sx0, sp0, sn0), (sx1, sp1, sn1))

    def issue(ci, b):
        sl = pl.ds(base + ci * _CROWS, _CROWS)
        return (
            pltpu.async_copy(x_hbm.at[sl, :], xb.at[b], sems[b][0]),
            pltpu.async_copy(p_hbm.at[sl, :], pb.at[b], sems[b][1]),
            pltpu.async_copy(n_hbm.at[sl, :], nb.at[b], sems[b][2]),
        )

    def compute(ci, b, carry):
        img = (base + ci * _CROWS) // _H
        iv = jnp.full((_L,), img, jnp.int32)
        lv = lbl.at[iv].get(mode="promise_in_bounds")
        is_pos = lv == 1
        sgn = jnp.where(is_pos, -1.0, 1.0)
        wgt = jnp.where(is_pos, _POS_WEIGHT, 1.0)

        def inner(i, c2):
            acc2, cnt2 = c2
            r = i // (_W // _L)
            sl = pl.ds((i % (_W // _L)) * _L, _L)
            x = xb[b, r, sl]
            p = pb[b, r, sl]
            n = nb[b, r, sl]
            t = x * sgn
            a = jnp.abs(t)
            rl = jnp.maximum(t, 0.0)
            u = jnp.exp(-a)
            poly = _C4
            for c in (_C3, _C2, _C1, _C0):
                poly = poly * u + c
            sp = rl + poly * u
            m = jnp.minimum(p, n) > 0.5
            acc2 = acc2 + jnp.where(m, sp, 0.0)
            cnt2 = cnt2 + jnp.where(m, 1.0, 0.0)
            return acc2, cnt2

        zero = jnp.zeros((_L,), jnp.float32)
        ca, cc = lax.fori_loop(0, _CVECS, inner, (zero, zero))
        acc, cnt = carry
        return acc + ca * wgt, cnt + cc

    zero = jnp.zeros((_L,), jnp.float32)
    acc, cnt = zero, zero
    pend = issue(0, 0)
    for ci in range(_NCHUNK):
        b = ci % 2
        nxt = issue(ci + 1, 1 - b) if ci + 1 < _NCHUNK else None
        for h in pend:
            h.wait()
        acc, cnt = compute(ci, b, (acc, cnt))
        pend = nxt
    av[...] = acc
    pltpu.sync_copy(av, out_hbm.at[0, wid])
    av[...] = cnt
    pltpu.sync_copy(av, out_hbm.at[1, wid])


def _tc_body(lbl_ref, x_ref, p_ref, n_ref, os_ref, oc_ref):
    i = pl.program_id(0)
    s_blk = 0.0
    c_blk = 0.0
    for h in range(2):
        y = lbl_ref[i * 2 + h]
        sgn = jnp.where(y == 1, -1.0, 1.0)
        wgt = jnp.where(y == 1, _POS_WEIGHT, 1.0)
        rows = pl.ds(h * _H, _H)
        x = x_ref[rows, :]
        t = x * sgn
        sp = jnp.maximum(t, 0.0) + jnp.log1p(jnp.exp(-jnp.abs(t)))
        m = (p_ref[rows, :] > 0.5) & (n_ref[rows, :] > 0.5)
        s_blk += jnp.sum(jnp.where(m, sp, 0.0)) * wgt
        c_blk += jnp.sum(jnp.where(m, 1.0, 0.0))

    @pl.when(i == 0)
    def _():
        os_ref[0, 0] = 0.0
        oc_ref[0, 0] = 0.0

    os_ref[0, 0] += s_blk
    oc_ref[0, 0] += c_blk


_tc_partial = pl.pallas_call(
    _tc_body,
    grid_spec=pltpu.PrefetchScalarGridSpec(
        num_scalar_prefetch=1,
        grid=(_TC_GRID,),
        in_specs=[
            pl.BlockSpec((_BR, _W), lambda i, *_: (i, 0)),
            pl.BlockSpec((_BR, _W), lambda i, *_: (i, 0)),
            pl.BlockSpec((_BR, _W), lambda i, *_: (i, 0)),
        ],
        out_specs=[
            pl.BlockSpec(memory_space=pltpu.SMEM),
            pl.BlockSpec(memory_space=pltpu.SMEM),
        ],
    ),
    out_shape=[
        jax.ShapeDtypeStruct((1, 1), jnp.float32),
        jax.ShapeDtypeStruct((1, 1), jnp.float32),
    ],
)


def _combine_body(parts_ref, ts_ref, tc_ref, out_ref):
    ps = parts_ref[...]
    num = jnp.sum(ps[0]) + ts_ref[0, 0]
    den = jnp.sum(ps[1]) + tc_ref[0, 0]
    out_ref[0, 0] = num / den


_combine = pl.pallas_call(
    _combine_body,
    in_specs=[
        pl.BlockSpec((2, _NW, _L), lambda: (0, 0, 0)),
        pl.BlockSpec(memory_space=pltpu.SMEM),
        pl.BlockSpec(memory_space=pltpu.SMEM),
    ],
    out_specs=pl.BlockSpec(memory_space=pltpu.SMEM),
    out_shape=jax.ShapeDtypeStruct((1, 1), jnp.float32),
)


def kernel(cancer_logits, prostate_mask, needle_mask, label, involvement):
    del involvement
    x = cancer_logits.reshape(_ROWS, _W)
    p = prostate_mask.reshape(_ROWS, _W)
    n = needle_mask.reshape(_ROWS, _W)
    lbl = label.astype(jnp.int32)
    parts = _sc_partial(x, p, n, lbl)
    tc_s, tc_c = _tc_partial(lbl, x, p, n)
    loss = _combine(parts, tc_s, tc_c)
    return loss[0, 0]
